# baseline (speedup 1.0000x reference)
import functools

import jax
import jax.numpy as jnp
from jax import lax
from jax.experimental import pallas as pl
from jax.experimental.pallas import tpu as pltpu
from jax.experimental.pallas import tpu_sc as plsc

_B, _L, _T1, _D = 4, 12, 257, 768
_E, _C, _IMG, _H = 8, 21, 224, 16
_NCORES, _NSUB = 2, 16
_LANES = 16


def _sc_route_body(cls_hbm, gwt_hbm, gb_hbm, out_hbm, cls_v, gw_v, gb_v, out_v):
    wid = lax.axis_index("s") * _NCORES + lax.axis_index("c")
    b = wid // _E
    e = wid % _E

    pltpu.sync_copy(cls_hbm.at[b], cls_v)
    pltpu.sync_copy(gwt_hbm.at[e], gw_v)
    pltpu.sync_copy(gb_hbm, gb_v)

    lane = lax.broadcasted_iota(jnp.int32, (_LANES,), 0)

    def dbody(d, accs):
        g = gw_v[pl.ds(d * _LANES, _LANES)]
        return tuple(accs[l] + cls_v[l, pl.ds(d * _LANES, _LANES)] * g
                     for l in range(_L))
    zero = jnp.zeros((_LANES,), jnp.float32)
    accs = lax.fori_loop(0, _D // _LANES, dbody, tuple(zero for _ in range(_L)))
    scores = zero
    for l in range(_L):
        scores = jnp.where(lane == l, jnp.sum(accs[l]), scores)
    gbe = jnp.sum(jnp.where(lane == e, gb_v[...], 0.0))
    scores = scores + gbe

    masked = jnp.where(lane < _L, scores, -3e38)
    m = jnp.max(masked)
    p = jnp.where(lane < _L, jnp.exp(masked - m), 0.0)
    prob = p / jnp.sum(p)

    v1 = jnp.max(prob)
    i1 = jnp.min(jnp.where(prob >= v1, lane, _LANES))
    prob2 = jnp.where(lane == i1, -1.0, prob)
    v2 = jnp.max(prob2)
    i2 = jnp.min(jnp.where(prob2 >= v2, lane, _LANES))

    t = jnp.exp(jnp.full((_LANES,), v2 - v1, jnp.float32))
    wa = 1.0 / (1.0 + t)
    wb = t * wa

    i1f = i1.astype(jnp.float32)
    i2f = i2.astype(jnp.float32)
    res = jnp.where(lane == 0, wa,
          jnp.where(lane == 1, wb,
          jnp.where(lane == 2, i1f,
          jnp.where(lane == 3, i2f, 0.0))))
    out_v[...] = res
    pltpu.sync_copy(out_v, out_hbm.at[wid])


@functools.cache
def _sc_route():
    return functools.partial(
        pl.kernel,
        out_type=jax.ShapeDtypeStruct((_B * _E, _LANES), jnp.float32),
        mesh=plsc.VectorSubcoreMesh(core_axis_name="c", subcore_axis_name="s",
                                    num_cores=_NCORES),
        compiler_params=pltpu.CompilerParams(needs_layout_passes=False),
        scratch_types=[
            pltpu.VMEM((_L, _D), jnp.float32),
            pltpu.VMEM((_D,), jnp.float32),
            pltpu.VMEM((_LANES,), jnp.float32),
            pltpu.VMEM((_LANES,), jnp.float32),
        ],
    )(_sc_route_body)


def _tc_body(isel_ref, wsel_ref, f_ref, w1_ref, b1_ref, w2_ref,
             b2_ref, out_ref, w1bf_ref):
    b = pl.program_id(0)

    @pl.when(b == 0)
    def _():
        for e in range(_E):
            w1bf_ref[e] = w1_ref[e].astype(jnp.bfloat16)

    acc = None
    for e in range(_E):
        k = b * _E + e
        la = f_ref[0, pl.ds(isel_ref[k, 0], 1)][0]
        lb = f_ref[0, pl.ds(isel_ref[k, 1], 1)][0]
        wa = wsel_ref[k, 0].astype(jnp.bfloat16)
        wb = wsel_ref[k, 1].astype(jnp.bfloat16)
        mixed = wa * la + wb * lb

        h = jnp.dot(mixed, w1bf_ref[e],
                    preferred_element_type=jnp.float32) + b1_ref[e]
        h = jnp.maximum(h, 0.0).astype(jnp.bfloat16)
        y_t = lax.dot_general(w2_ref[e], h, (((0,), (1,)), ((), ())),
                              preferred_element_type=jnp.float32)
        y_t = (y_t + b2_ref[e]) * (1.0 / _E)
        acc = y_t if acc is None else acc + y_t

    out_ref[0] = acc[:, 1:]


def _resize_body(avg_ref, r_ref, rt_ref, out_ref):
    a1 = jnp.dot(avg_ref[0], rt_ref[...],
                 preferred_element_type=jnp.float32)
    r = r_ref[...]
    for c in range(_C):
        out_ref[0, c] = jnp.dot(r, a1[c * _H:(c + 1) * _H, :],
                                preferred_element_type=jnp.float32)


def kernel(features, gate_w, gate_b, w1, b1, w2, b2):
    B, L, T1, D = features.shape
    E = w1.shape[0]
    C = w2.shape[2]
    T = T1 - 1

    cls = features[:, :, 0, :]
    gwt = gate_w.T
    gb16 = jnp.pad(gate_b, (0, _LANES - E))
    sel = _sc_route()(cls, gwt, gb16)
    wsel = sel[:, 0:2]
    isel = sel[:, 2:4].astype(jnp.int32)

    f_bf = features.astype(jnp.bfloat16)
    w2_bf = w2.astype(jnp.bfloat16)
    grid_spec = pltpu.PrefetchScalarGridSpec(
        num_scalar_prefetch=2,
        grid=(B,),
        in_specs=[
            pl.BlockSpec((1, L, T1, D), lambda b, i_s, w_s: (b, 0, 0, 0)),
            pl.BlockSpec((E, D, D), lambda b, i_s, w_s: (0, 0, 0)),
            pl.BlockSpec((E, 1, D), lambda b, i_s, w_s: (0, 0, 0)),
            pl.BlockSpec((E, D, C), lambda b, i_s, w_s: (0, 0, 0)),
            pl.BlockSpec((E, C, 1), lambda b, i_s, w_s: (0, 0, 0)),
        ],
        out_specs=pl.BlockSpec((1, C, T), lambda b, i_s, w_s: (b, 0, 0)),
        scratch_shapes=[pltpu.VMEM((E, D, D), jnp.bfloat16)],
    )
    avg = pl.pallas_call(
        _tc_body,
        grid_spec=grid_spec,
        out_shape=jax.ShapeDtypeStruct((B, C, T), jnp.float32),
    )(isel, wsel, f_bf, w1, b1.reshape(E, 1, D),
      w2_bf, b2.reshape(E, C, 1))

    avg_m = avg.reshape(B, C * _H, _H)

    r_mat = jax.image.resize(jnp.eye(_H, dtype=jnp.float32), (_IMG, _H),
                             method="bilinear")

    out = pl.pallas_call(
        _resize_body,
        grid=(B,),
        in_specs=[
            pl.BlockSpec((1, C * _H, _H), lambda b: (b, 0, 0)),
            pl.BlockSpec((_IMG, _H), lambda b: (0, 0)),
            pl.BlockSpec((_H, _IMG), lambda b: (0, 0)),
        ],
        out_specs=pl.BlockSpec((1, C, _IMG, _IMG), lambda b: (b, 0, 0, 0)),
        out_shape=jax.ShapeDtypeStruct((B, C, _IMG, _IMG), jnp.float32),
    )(avg_m, r_mat, r_mat.T)
    return out


# speedup vs baseline: 1.0230x; 1.0230x over previous
import functools

import jax
import jax.numpy as jnp
from jax import lax
from jax.experimental import pallas as pl
from jax.experimental.pallas import tpu as pltpu
from jax.experimental.pallas import tpu_sc as plsc

_B, _L, _T1, _D = 4, 12, 257, 768
_E, _C, _IMG, _H = 8, 21, 224, 16
_NCORES, _NSUB = 2, 16
_LANES = 16


def _sc_route_body(cls_hbm, gwt_hbm, gb_hbm, out_hbm, cls_v, gw_v, gb_v, out_v):
    wid = lax.axis_index("s") * _NCORES + lax.axis_index("c")
    b = wid // _E
    e = wid % _E

    pltpu.sync_copy(cls_hbm.at[b], cls_v)
    pltpu.sync_copy(gwt_hbm.at[e], gw_v)
    pltpu.sync_copy(gb_hbm, gb_v)

    lane = lax.broadcasted_iota(jnp.int32, (_LANES,), 0)

    def dbody(d, accs):
        g = gw_v[pl.ds(d * _LANES, _LANES)]
        return tuple(accs[l] + cls_v[l, pl.ds(d * _LANES, _LANES)] * g
                     for l in range(_L))
    zero = jnp.zeros((_LANES,), jnp.float32)
    accs = lax.fori_loop(0, _D // _LANES, dbody, tuple(zero for _ in range(_L)))
    scores = zero
    for l in range(_L):
        scores = jnp.where(lane == l, jnp.sum(accs[l]), scores)
    gbe = jnp.sum(jnp.where(lane == e, gb_v[...], 0.0))
    scores = scores + gbe

    masked = jnp.where(lane < _L, scores, -3e38)
    m = jnp.max(masked)
    p = jnp.where(lane < _L, jnp.exp(masked - m), 0.0)
    prob = p / jnp.sum(p)

    v1 = jnp.max(prob)
    i1 = jnp.min(jnp.where(prob >= v1, lane, _LANES))
    prob2 = jnp.where(lane == i1, -1.0, prob)
    v2 = jnp.max(prob2)
    i2 = jnp.min(jnp.where(prob2 >= v2, lane, _LANES))

    t = jnp.exp(jnp.full((_LANES,), v2 - v1, jnp.float32))
    wa = 1.0 / (1.0 + t)
    wb = t * wa

    i1f = i1.astype(jnp.float32)
    i2f = i2.astype(jnp.float32)
    res = jnp.where(lane == 0, wa,
          jnp.where(lane == 1, wb,
          jnp.where(lane == 2, i1f,
          jnp.where(lane == 3, i2f, 0.0))))
    out_v[...] = res
    pltpu.sync_copy(out_v, out_hbm.at[wid])


@functools.cache
def _sc_route():
    return functools.partial(
        pl.kernel,
        out_type=jax.ShapeDtypeStruct((_B * _E, _LANES), jnp.float32),
        mesh=plsc.VectorSubcoreMesh(core_axis_name="c", subcore_axis_name="s",
                                    num_cores=_NCORES),
        compiler_params=pltpu.CompilerParams(needs_layout_passes=False),
        scratch_types=[
            pltpu.VMEM((_L, _D), jnp.float32),
            pltpu.VMEM((_D,), jnp.float32),
            pltpu.VMEM((_LANES,), jnp.float32),
            pltpu.VMEM((_LANES,), jnp.float32),
        ],
    )(_sc_route_body)


def _tc_body(isel_ref, wsel_ref, f_ref, w1_ref, b1_ref, w2_ref,
             b2_ref, out_ref, w1bf_ref):
    b = pl.program_id(0)

    @pl.when(b == 0)
    def _():
        for e in range(_E):
            w1bf_ref[e] = w1_ref[e].astype(jnp.bfloat16)

    acc = None
    for e in range(_E):
        k = b * _E + e
        la = f_ref[0, pl.ds(isel_ref[k, 0], 1)][0]
        lb = f_ref[0, pl.ds(isel_ref[k, 1], 1)][0]
        mixed = wsel_ref[k, 0] * la + wsel_ref[k, 1] * lb
        mixed = mixed.astype(jnp.bfloat16)

        h = jnp.dot(mixed, w1bf_ref[e],
                    preferred_element_type=jnp.float32).astype(jnp.bfloat16)
        h = jnp.maximum(h + b1_ref[e], jnp.bfloat16(0.0))
        y_t = lax.dot_general(w2_ref[e], h, (((0,), (1,)), ((), ())),
                              preferred_element_type=jnp.float32)
        y_t = (y_t + b2_ref[e]) * (1.0 / _E)
        acc = y_t if acc is None else acc + y_t

    out_ref[0] = acc[:, 1:]


def _resize_body(avg_ref, r_ref, rt_ref, out_ref):
    a1 = jnp.dot(avg_ref[0], rt_ref[...],
                 preferred_element_type=jnp.float32)
    r = r_ref[...]
    for c in range(_C):
        out_ref[0, c] = jnp.dot(r, a1[c * _H:(c + 1) * _H, :],
                                preferred_element_type=jnp.float32)


def kernel(features, gate_w, gate_b, w1, b1, w2, b2):
    B, L, T1, D = features.shape
    E = w1.shape[0]
    C = w2.shape[2]
    T = T1 - 1

    cls = features[:, :, 0, :]
    gwt = gate_w.T
    gb16 = jnp.pad(gate_b, (0, _LANES - E))
    sel = _sc_route()(cls, gwt, gb16)
    wsel = sel[:, 0:2]
    isel = sel[:, 2:4].astype(jnp.int32)

    w2_bf = w2.astype(jnp.bfloat16)
    grid_spec = pltpu.PrefetchScalarGridSpec(
        num_scalar_prefetch=2,
        grid=(B,),
        in_specs=[
            pl.BlockSpec((1, L, T1, D), lambda b, i_s, w_s: (b, 0, 0, 0)),
            pl.BlockSpec((E, D, D), lambda b, i_s, w_s: (0, 0, 0)),
            pl.BlockSpec((E, 1, D), lambda b, i_s, w_s: (0, 0, 0)),
            pl.BlockSpec((E, D, C), lambda b, i_s, w_s: (0, 0, 0)),
            pl.BlockSpec((E, C, 1), lambda b, i_s, w_s: (0, 0, 0)),
        ],
        out_specs=pl.BlockSpec((1, C, T), lambda b, i_s, w_s: (b, 0, 0)),
        scratch_shapes=[pltpu.VMEM((E, D, D), jnp.bfloat16)],
    )
    avg = pl.pallas_call(
        _tc_body,
        grid_spec=grid_spec,
        out_shape=jax.ShapeDtypeStruct((B, C, T), jnp.float32),
    )(isel, wsel, features, w1, b1.reshape(E, 1, D).astype(jnp.bfloat16),
      w2_bf, b2.reshape(E, C, 1))

    avg_m = avg.reshape(B, C * _H, _H)

    r_mat = jax.image.resize(jnp.eye(_H, dtype=jnp.float32), (_IMG, _H),
                             method="bilinear")

    out = pl.pallas_call(
        _resize_body,
        grid=(B,),
        in_specs=[
            pl.BlockSpec((1, C * _H, _H), lambda b: (b, 0, 0)),
            pl.BlockSpec((_IMG, _H), lambda b: (0, 0)),
            pl.BlockSpec((_H, _IMG), lambda b: (0, 0)),
        ],
        out_specs=pl.BlockSpec((1, C, _IMG, _IMG), lambda b: (b, 0, 0, 0)),
        out_shape=jax.ShapeDtypeStruct((B, C, _IMG, _IMG), jnp.float32),
    )(avg_m, r_mat, r_mat.T)
    return out


# speedup vs baseline: 1.1081x; 1.0832x over previous
import functools

import jax
import jax.numpy as jnp
from jax import lax
from jax.experimental import pallas as pl
from jax.experimental.pallas import tpu as pltpu
from jax.experimental.pallas import tpu_sc as plsc

_B, _L, _T1, _D = 4, 12, 257, 768
_E, _C, _IMG, _H = 8, 21, 224, 16
_NCORES, _NSUB = 2, 16
_LANES = 16


def _sc_route_body(cls_hbm, gwt_hbm, gb_hbm, out_hbm, cls_v, gw_v, gb_v, out_v):
    wid = lax.axis_index("s") * _NCORES + lax.axis_index("c")
    b = wid // _E
    e = wid % _E

    pltpu.sync_copy(cls_hbm.at[b], cls_v)
    pltpu.sync_copy(gwt_hbm.at[e], gw_v)
    pltpu.sync_copy(gb_hbm, gb_v)

    lane = lax.broadcasted_iota(jnp.int32, (_LANES,), 0)

    def dbody(d, accs):
        g = gw_v[pl.ds(d * _LANES, _LANES)]
        return tuple(accs[l] + cls_v[l, pl.ds(d * _LANES, _LANES)] * g
                     for l in range(_L))
    zero = jnp.zeros((_LANES,), jnp.float32)
    accs = lax.fori_loop(0, _D // _LANES, dbody, tuple(zero for _ in range(_L)))
    scores = zero
    for l in range(_L):
        scores = jnp.where(lane == l, jnp.sum(accs[l]), scores)
    gbe = jnp.sum(jnp.where(lane == e, gb_v[...], 0.0))
    scores = scores + gbe

    masked = jnp.where(lane < _L, scores, -3e38)
    m = jnp.max(masked)
    p = jnp.where(lane < _L, jnp.exp(masked - m), 0.0)
    prob = p / jnp.sum(p)

    v1 = jnp.max(prob)
    i1 = jnp.min(jnp.where(prob >= v1, lane, _LANES))
    prob2 = jnp.where(lane == i1, -1.0, prob)
    v2 = jnp.max(prob2)
    i2 = jnp.min(jnp.where(prob2 >= v2, lane, _LANES))

    t = jnp.exp(jnp.full((_LANES,), v2 - v1, jnp.float32))
    wa = 1.0 / (1.0 + t)
    wb = t * wa

    i1f = i1.astype(jnp.float32)
    i2f = i2.astype(jnp.float32)
    res = jnp.where(lane == 0, wa,
          jnp.where(lane == 1, wb,
          jnp.where(lane == 2, i1f,
          jnp.where(lane == 3, i2f, 0.0))))
    out_v[...] = res
    pltpu.sync_copy(out_v, out_hbm.at[wid])


@functools.cache
def _sc_route():
    return functools.partial(
        pl.kernel,
        out_type=jax.ShapeDtypeStruct((_B * _E, _LANES), jnp.float32),
        mesh=plsc.VectorSubcoreMesh(core_axis_name="c", subcore_axis_name="s",
                                    num_cores=_NCORES),
        compiler_params=pltpu.CompilerParams(needs_layout_passes=False),
        scratch_types=[
            pltpu.VMEM((_L, _D), jnp.float32),
            pltpu.VMEM((_D,), jnp.float32),
            pltpu.VMEM((_LANES,), jnp.float32),
            pltpu.VMEM((_LANES,), jnp.float32),
        ],
    )(_sc_route_body)


def _tc_body(isel_ref, wsel_ref, f_ref, w1_ref, b1_ref, w2_ref,
             b2_ref, out_ref, w1bf_ref):
    b = pl.program_id(0)

    @pl.when(b == 0)
    def _():
        for e in range(_E):
            w1bf_ref[e] = w1_ref[e].astype(jnp.bfloat16)

    acc = None
    for e in range(_E):
        k = b * _E + e
        la = f_ref[0, pl.ds(isel_ref[k, 0], 1)][0]
        lb = f_ref[0, pl.ds(isel_ref[k, 1], 1)][0]
        mixed = wsel_ref[k, 0] * la + wsel_ref[k, 1] * lb
        mixed = mixed.astype(jnp.bfloat16)

        h = jnp.dot(mixed, w1bf_ref[e],
                    preferred_element_type=jnp.float32) + b1_ref[e]
        h = jnp.maximum(h, 0.0).astype(jnp.bfloat16)
        y = jnp.dot(h, w2_ref[e],
                    preferred_element_type=jnp.float32)
        acc = y if acc is None else acc + y

    bsum = jnp.sum(b2_ref[...], axis=0)
    out_ref[0] = acc[1:, :] * (1.0 / _E) + bsum * (1.0 / _E)


def _resize_body(avg_ref, r_ref, rt_ref, out_ref):
    a1 = jnp.dot(avg_ref[0], rt_ref[...],
                 preferred_element_type=jnp.float32)
    r = r_ref[...]
    for c in range(_C):
        out_ref[0, c] = jnp.dot(r, a1[c * _H:(c + 1) * _H, :],
                                preferred_element_type=jnp.float32)


def kernel(features, gate_w, gate_b, w1, b1, w2, b2):
    B, L, T1, D = features.shape
    E = w1.shape[0]
    C = w2.shape[2]
    T = T1 - 1

    cls = features[:, :, 0, :]
    gwt = gate_w.T
    gb16 = jnp.pad(gate_b, (0, _LANES - E))
    sel = _sc_route()(cls, gwt, gb16)
    wsel = sel[:, 0:2]
    isel = sel[:, 2:4].astype(jnp.int32)

    w2_bf = w2.astype(jnp.bfloat16)
    grid_spec = pltpu.PrefetchScalarGridSpec(
        num_scalar_prefetch=2,
        grid=(B,),
        in_specs=[
            pl.BlockSpec((1, L, T1, D), lambda b, i_s, w_s: (b, 0, 0, 0)),
            pl.BlockSpec((E, D, D), lambda b, i_s, w_s: (0, 0, 0)),
            pl.BlockSpec((E, 1, D), lambda b, i_s, w_s: (0, 0, 0)),
            pl.BlockSpec((E, D, C), lambda b, i_s, w_s: (0, 0, 0)),
            pl.BlockSpec((E, 1, C), lambda b, i_s, w_s: (0, 0, 0)),
        ],
        out_specs=pl.BlockSpec((1, T, C), lambda b, i_s, w_s: (b, 0, 0)),
        scratch_shapes=[pltpu.VMEM((E, D, D), jnp.bfloat16)],
    )
    avg = pl.pallas_call(
        _tc_body,
        grid_spec=grid_spec,
        out_shape=jax.ShapeDtypeStruct((B, T, C), jnp.float32),
    )(isel, wsel, features, w1, b1.reshape(E, 1, D),
      w2_bf, b2.reshape(E, 1, C))

    avg_m = avg.transpose(0, 2, 1).reshape(B, C * _H, _H)

    r_mat = jax.image.resize(jnp.eye(_H, dtype=jnp.float32), (_IMG, _H),
                             method="bilinear")

    out = pl.pallas_call(
        _resize_body,
        grid=(B,),
        in_specs=[
            pl.BlockSpec((1, C * _H, _H), lambda b: (b, 0, 0)),
            pl.BlockSpec((_IMG, _H), lambda b: (0, 0)),
            pl.BlockSpec((_H, _IMG), lambda b: (0, 0)),
        ],
        out_specs=pl.BlockSpec((1, C, _IMG, _IMG), lambda b: (b, 0, 0, 0)),
        out_shape=jax.ShapeDtypeStruct((B, C, _IMG, _IMG), jnp.float32),
    )(avg_m, r_mat, r_mat.T)
    return out


# speedup vs baseline: 1.1109x; 1.0025x over previous
import functools

import jax
import jax.numpy as jnp
from jax import lax
from jax.experimental import pallas as pl
from jax.experimental.pallas import tpu as pltpu
from jax.experimental.pallas import tpu_sc as plsc

_B, _L, _T1, _D = 4, 12, 257, 768
_E, _C, _IMG, _H = 8, 21, 224, 16
_NCORES, _NSUB = 2, 16
_LANES = 16


def _sc_route_body(cls_hbm, gwt_hbm, gb_hbm, out_hbm, cls_v, gw_v, gb_v, out_v):
    wid = lax.axis_index("s") * _NCORES + lax.axis_index("c")
    b = wid // _E
    e = wid % _E

    pltpu.sync_copy(cls_hbm.at[b], cls_v)
    pltpu.sync_copy(gwt_hbm.at[e], gw_v)
    pltpu.sync_copy(gb_hbm, gb_v)

    lane = lax.broadcasted_iota(jnp.int32, (_LANES,), 0)

    def dbody(d, accs):
        g = gw_v[pl.ds(d * _LANES, _LANES)]
        return tuple(accs[l] + cls_v[l, pl.ds(d * _LANES, _LANES)] * g
                     for l in range(_L))
    zero = jnp.zeros((_LANES,), jnp.float32)
    accs = lax.fori_loop(0, _D // _LANES, dbody, tuple(zero for _ in range(_L)))
    scores = zero
    for l in range(_L):
        scores = jnp.where(lane == l, jnp.sum(accs[l]), scores)
    gbe = jnp.sum(jnp.where(lane == e, gb_v[...], 0.0))
    scores = scores + gbe

    masked = jnp.where(lane < _L, scores, -3e38)
    m = jnp.max(masked)
    p = jnp.where(lane < _L, jnp.exp(masked - m), 0.0)
    prob = p / jnp.sum(p)

    v1 = jnp.max(prob)
    i1 = jnp.min(jnp.where(prob >= v1, lane, _LANES))
    prob2 = jnp.where(lane == i1, -1.0, prob)
    v2 = jnp.max(prob2)
    i2 = jnp.min(jnp.where(prob2 >= v2, lane, _LANES))

    t = jnp.exp(jnp.full((_LANES,), v2 - v1, jnp.float32))
    wa = 1.0 / (1.0 + t)
    wb = t * wa

    i1f = i1.astype(jnp.float32)
    i2f = i2.astype(jnp.float32)
    res = jnp.where(lane == 0, wa,
          jnp.where(lane == 1, wb,
          jnp.where(lane == 2, i1f,
          jnp.where(lane == 3, i2f, 0.0))))
    out_v[...] = res
    pltpu.sync_copy(out_v, out_hbm.at[wid])


@functools.cache
def _sc_route():
    return functools.partial(
        pl.kernel,
        out_type=jax.ShapeDtypeStruct((_B * _E, _LANES), jnp.float32),
        mesh=plsc.VectorSubcoreMesh(core_axis_name="c", subcore_axis_name="s",
                                    num_cores=_NCORES),
        compiler_params=pltpu.CompilerParams(needs_layout_passes=False),
        scratch_types=[
            pltpu.VMEM((_L, _D), jnp.float32),
            pltpu.VMEM((_D,), jnp.float32),
            pltpu.VMEM((_LANES,), jnp.float32),
            pltpu.VMEM((_LANES,), jnp.float32),
        ],
    )(_sc_route_body)


def _tc_body(isel_ref, wsel_ref, f_ref, w1_ref, b1_ref, w2_ref,
             b2_ref, out_ref, w1bf_ref):
    b = pl.program_id(0)

    @pl.when(b == 0)
    def _():
        for e in range(_E):
            w1bf_ref[e] = w1_ref[e].astype(jnp.bfloat16)

    acc = None
    for e in range(_E):
        k = b * _E + e
        la = f_ref[0, pl.ds(isel_ref[k, 0], 1)][0]
        lb = f_ref[0, pl.ds(isel_ref[k, 1], 1)][0]
        mixed = wsel_ref[k, 0] * la + wsel_ref[k, 1] * lb
        mixed = mixed.astype(jnp.bfloat16)

        h = jnp.dot(mixed, w1bf_ref[e],
                    preferred_element_type=jnp.float32) + b1_ref[e]
        h = jnp.maximum(h, 0.0).astype(jnp.bfloat16)
        y = jnp.dot(h, w2_ref[e],
                    preferred_element_type=jnp.float32)
        acc = y if acc is None else acc + y

    bsum = jnp.sum(b2_ref[...], axis=0)
    out_ref[0] = acc[1:, :] * (1.0 / _E) + bsum * (1.0 / _E)


def _resize_body(avg_ref, r_ref, rt_ref, out_ref):
    a1 = jnp.dot(avg_ref[0], rt_ref[...],
                 preferred_element_type=jnp.float32)
    r = r_ref[...]
    for c in range(_C):
        out_ref[0, c] = jnp.dot(r, a1[c * _H:(c + 1) * _H, :],
                                preferred_element_type=jnp.float32)


def kernel(features, gate_w, gate_b, w1, b1, w2, b2):
    B, L, T1, D = features.shape
    E = w1.shape[0]
    C = w2.shape[2]
    T = T1 - 1

    cls = features[:, :, 0, :]
    gwt = gate_w.T
    gb16 = jnp.pad(gate_b, (0, _LANES - E))
    sel = _sc_route()(cls, gwt, gb16)
    wsel = sel[:, 0:2]
    isel = sel[:, 2:4].astype(jnp.int32)

    w2_bf = w2.astype(jnp.bfloat16)
    grid_spec = pltpu.PrefetchScalarGridSpec(
        num_scalar_prefetch=2,
        grid=(B,),
        in_specs=[
            pl.BlockSpec((1, L, T1, D), lambda b, i_s, w_s: (b, 0, 0, 0)),
            pl.BlockSpec((E, D, D), lambda b, i_s, w_s: (0, 0, 0)),
            pl.BlockSpec((E, 1, D), lambda b, i_s, w_s: (0, 0, 0)),
            pl.BlockSpec((E, D, C), lambda b, i_s, w_s: (0, 0, 0)),
            pl.BlockSpec((E, 1, C), lambda b, i_s, w_s: (0, 0, 0)),
        ],
        out_specs=pl.BlockSpec((1, T, C), lambda b, i_s, w_s: (b, 0, 0)),
        scratch_shapes=[pltpu.VMEM((E, D, D), jnp.bfloat16)],
    )
    avg = pl.pallas_call(
        _tc_body,
        grid_spec=grid_spec,
        out_shape=jax.ShapeDtypeStruct((B, T, C), jnp.float32),
        compiler_params=pltpu.CompilerParams(
            allow_input_fusion=[True, False, False, False, False]),
    )(isel, wsel, features, w1, b1.reshape(E, 1, D),
      w2_bf, b2.reshape(E, 1, C))

    avg_m = avg.transpose(0, 2, 1).reshape(B, C * _H, _H)

    r_mat = jax.image.resize(jnp.eye(_H, dtype=jnp.float32), (_IMG, _H),
                             method="bilinear")

    out = pl.pallas_call(
        _resize_body,
        grid=(B,),
        in_specs=[
            pl.BlockSpec((1, C * _H, _H), lambda b: (b, 0, 0)),
            pl.BlockSpec((_IMG, _H), lambda b: (0, 0)),
            pl.BlockSpec((_H, _IMG), lambda b: (0, 0)),
        ],
        out_specs=pl.BlockSpec((1, C, _IMG, _IMG), lambda b: (b, 0, 0, 0)),
        out_shape=jax.ShapeDtypeStruct((B, C, _IMG, _IMG), jnp.float32),
    )(avg_m, r_mat, r_mat.T)
    return out


# speedup vs baseline: 1.1154x; 1.0040x over previous
import functools

import jax
import jax.numpy as jnp
from jax import lax
from jax.experimental import pallas as pl
from jax.experimental.pallas import tpu as pltpu
from jax.experimental.pallas import tpu_sc as plsc

_B, _L, _T1, _D = 4, 12, 257, 768
_E, _C, _IMG, _H = 8, 21, 224, 16
_NCORES, _NSUB = 2, 16
_LANES = 16


def _sc_route_body(cls_hbm, gwt_hbm, gb_hbm, out_hbm, cls_v, gw_v, gb_v, out_v):
    wid = lax.axis_index("s") * _NCORES + lax.axis_index("c")
    b = wid // _E
    e = wid % _E

    pltpu.sync_copy(cls_hbm.at[b], cls_v)
    pltpu.sync_copy(gwt_hbm.at[e], gw_v)
    pltpu.sync_copy(gb_hbm, gb_v)

    lane = lax.broadcasted_iota(jnp.int32, (_LANES,), 0)

    def dbody(d, accs):
        g = gw_v[pl.ds(d * _LANES, _LANES)]
        return tuple(accs[l] + cls_v[l, pl.ds(d * _LANES, _LANES)] * g
                     for l in range(_L))
    zero = jnp.zeros((_LANES,), jnp.float32)
    accs = lax.fori_loop(0, _D // _LANES, dbody, tuple(zero for _ in range(_L)))
    scores = zero
    for l in range(_L):
        scores = jnp.where(lane == l, jnp.sum(accs[l]), scores)
    gbe = jnp.sum(jnp.where(lane == e, gb_v[...], 0.0))
    scores = scores + gbe

    masked = jnp.where(lane < _L, scores, -3e38)
    m = jnp.max(masked)
    p = jnp.where(lane < _L, jnp.exp(masked - m), 0.0)
    prob = p / jnp.sum(p)

    v1 = jnp.max(prob)
    i1 = jnp.min(jnp.where(prob >= v1, lane, _LANES))
    prob2 = jnp.where(lane == i1, -1.0, prob)
    v2 = jnp.max(prob2)
    i2 = jnp.min(jnp.where(prob2 >= v2, lane, _LANES))

    t = jnp.exp(jnp.full((_LANES,), v2 - v1, jnp.float32))
    wa = 1.0 / (1.0 + t)
    wb = t * wa

    i1f = i1.astype(jnp.float32)
    i2f = i2.astype(jnp.float32)
    res = jnp.where(lane == 0, wa,
          jnp.where(lane == 1, wb,
          jnp.where(lane == 2, i1f,
          jnp.where(lane == 3, i2f, 0.0))))
    out_v[...] = res
    pltpu.sync_copy(out_v, out_hbm.at[wid])


@functools.cache
def _sc_route():
    return functools.partial(
        pl.kernel,
        out_type=jax.ShapeDtypeStruct((_B * _E, _LANES), jnp.float32),
        mesh=plsc.VectorSubcoreMesh(core_axis_name="c", subcore_axis_name="s",
                                    num_cores=_NCORES),
        compiler_params=pltpu.CompilerParams(needs_layout_passes=False),
        scratch_types=[
            pltpu.VMEM((_L, _D), jnp.float32),
            pltpu.VMEM((_D,), jnp.float32),
            pltpu.VMEM((_LANES,), jnp.float32),
            pltpu.VMEM((_LANES,), jnp.float32),
        ],
    )(_sc_route_body)


def _tc_body(isel_ref, wsel_ref, f_ref, w1_ref, b1_ref, w2_ref,
             b2_ref, out_ref, w1bf_ref):
    b = pl.program_id(0)

    @pl.when(b == 0)
    def _():
        for e in range(_E):
            w1bf_ref[e] = w1_ref[e].astype(jnp.bfloat16)

    acc = None
    for e in range(_E):
        k = b * _E + e
        la = f_ref[0, pl.ds(isel_ref[k, 0], 1)][0]
        lb = f_ref[0, pl.ds(isel_ref[k, 1], 1)][0]
        mixed = wsel_ref[k, 0] * la + wsel_ref[k, 1] * lb
        mixed = mixed.astype(jnp.bfloat16)

        h = jnp.dot(mixed, w1bf_ref[e],
                    preferred_element_type=jnp.float32) + b1_ref[e]
        h = jnp.maximum(h, 0.0).astype(jnp.bfloat16)
        y = jnp.dot(h, w2_ref[e],
                    preferred_element_type=jnp.float32)
        acc = y if acc is None else acc + y

    bsum = jnp.sum(b2_ref[...], axis=0)
    out_ref[0] = acc[1:, :] * (1.0 / _E) + bsum * (1.0 / _E)


def _resize_body(avg_ref, r_ref, rt_ref, out_ref):
    a1 = jnp.dot(avg_ref[0], rt_ref[...],
                 preferred_element_type=jnp.float32)
    r = r_ref[...]
    for c in range(_C):
        out_ref[0, c] = jnp.dot(r, a1[c * _H:(c + 1) * _H, :],
                                preferred_element_type=jnp.float32)


def kernel(features, gate_w, gate_b, w1, b1, w2, b2):
    B, L, T1, D = features.shape
    E = w1.shape[0]
    C = w2.shape[2]
    T = T1 - 1

    cls = features[:, :, 0, :]
    gwt = gate_w.T
    gb16 = jnp.pad(gate_b, (0, _LANES - E))
    sel = _sc_route()(cls, gwt, gb16)
    wsel = sel[:, 0:2]
    isel = sel[:, 2:4].astype(jnp.int32)

    w2_bf = w2.astype(jnp.bfloat16)
    grid_spec = pltpu.PrefetchScalarGridSpec(
        num_scalar_prefetch=2,
        grid=(B,),
        in_specs=[
            pl.BlockSpec((1, L, T1, D), lambda b, i_s, w_s: (b, 0, 0, 0)),
            pl.BlockSpec((E, D, D), lambda b, i_s, w_s: (0, 0, 0)),
            pl.BlockSpec((E, 1, D), lambda b, i_s, w_s: (0, 0, 0)),
            pl.BlockSpec((E, D, C), lambda b, i_s, w_s: (0, 0, 0)),
            pl.BlockSpec((E, 1, C), lambda b, i_s, w_s: (0, 0, 0)),
        ],
        out_specs=pl.BlockSpec((1, T, C), lambda b, i_s, w_s: (b, 0, 0)),
        scratch_shapes=[pltpu.VMEM((E, D, D), jnp.bfloat16)],
    )
    avg = pl.pallas_call(
        _tc_body,
        grid_spec=grid_spec,
        out_shape=jax.ShapeDtypeStruct((B, T, C), jnp.float32),
    )(isel, wsel, features, w1, b1.reshape(E, 1, D),
      w2_bf, b2.reshape(E, 1, C))

    avg_m = avg.transpose(0, 2, 1).reshape(B, C * _H, _H)

    r_mat = jax.image.resize(jnp.eye(_H, dtype=jnp.float32), (_IMG, _H),
                             method="bilinear")

    out = pl.pallas_call(
        _resize_body,
        grid=(B,),
        in_specs=[
            pl.BlockSpec((1, C * _H, _H), lambda b: (b, 0, 0)),
            pl.BlockSpec((_IMG, _H), lambda b: (0, 0)),
            pl.BlockSpec((_H, _IMG), lambda b: (0, 0)),
        ],
        out_specs=pl.BlockSpec((1, C, _IMG, _IMG), lambda b: (b, 0, 0, 0)),
        out_shape=jax.ShapeDtypeStruct((B, C, _IMG, _IMG), jnp.float32),
    )(avg_m, r_mat, r_mat.T)
    return out


# speedup vs baseline: 1.1191x; 1.0034x over previous
import functools

import jax
import jax.numpy as jnp
from jax import lax
from jax.experimental import pallas as pl
from jax.experimental.pallas import tpu as pltpu
from jax.experimental.pallas import tpu_sc as plsc

_B, _L, _T1, _D = 4, 12, 257, 768
_E, _C, _IMG, _H = 8, 21, 224, 16
_NCORES, _NSUB = 2, 16
_LANES = 16


def _sc_route_body(cls_hbm, gwt_hbm, gb_hbm, out_hbm, cls_v, gw_v, gb_v, out_v):
    wid = lax.axis_index("s") * _NCORES + lax.axis_index("c")
    b = wid // _E
    e = wid % _E

    pltpu.sync_copy(cls_hbm.at[b], cls_v)
    pltpu.sync_copy(gwt_hbm.at[e], gw_v)
    pltpu.sync_copy(gb_hbm, gb_v)

    lane = lax.broadcasted_iota(jnp.int32, (_LANES,), 0)

    def dbody(d, accs):
        g = gw_v[pl.ds(d * _LANES, _LANES)]
        return tuple(accs[l] + cls_v[l, pl.ds(d * _LANES, _LANES)] * g
                     for l in range(_L))
    zero = jnp.zeros((_LANES,), jnp.float32)
    accs = lax.fori_loop(0, _D // _LANES, dbody, tuple(zero for _ in range(_L)))
    scores = zero
    for l in range(_L):
        scores = jnp.where(lane == l, jnp.sum(accs[l]), scores)
    gbe = jnp.sum(jnp.where(lane == e, gb_v[...], 0.0))
    scores = scores + gbe

    masked = jnp.where(lane < _L, scores, -3e38)
    m = jnp.max(masked)
    p = jnp.where(lane < _L, jnp.exp(masked - m), 0.0)
    prob = p / jnp.sum(p)

    v1 = jnp.max(prob)
    i1 = jnp.min(jnp.where(prob >= v1, lane, _LANES))
    prob2 = jnp.where(lane == i1, -1.0, prob)
    v2 = jnp.max(prob2)
    i2 = jnp.min(jnp.where(prob2 >= v2, lane, _LANES))

    t = jnp.exp(jnp.full((_LANES,), v2 - v1, jnp.float32))
    wa = 1.0 / (1.0 + t)
    wb = t * wa

    i1f = i1.astype(jnp.float32)
    i2f = i2.astype(jnp.float32)
    res = jnp.where(lane == 0, wa,
          jnp.where(lane == 1, wb,
          jnp.where(lane == 2, i1f,
          jnp.where(lane == 3, i2f, 0.0))))
    out_v[...] = res
    pltpu.sync_copy(out_v, out_hbm.at[wid])


@functools.cache
def _sc_route():
    return functools.partial(
        pl.kernel,
        out_type=jax.ShapeDtypeStruct((_B * _E, _LANES), jnp.float32),
        mesh=plsc.VectorSubcoreMesh(core_axis_name="c", subcore_axis_name="s",
                                    num_cores=_NCORES),
        compiler_params=pltpu.CompilerParams(needs_layout_passes=False),
        scratch_types=[
            pltpu.VMEM((_L, _D), jnp.float32),
            pltpu.VMEM((_D,), jnp.float32),
            pltpu.VMEM((_LANES,), jnp.float32),
            pltpu.VMEM((_LANES,), jnp.float32),
        ],
    )(_sc_route_body)


def _tc_body(sel_ref, f_ref, w1_ref, b1_ref, w2_ref,
             b2_ref, out_ref, w1bf_ref):
    b = pl.program_id(0)

    @pl.when(b == 0)
    def _():
        for e in range(_E):
            w1bf_ref[e] = w1_ref[e].astype(jnp.bfloat16)

    acc = None
    for e in range(_E):
        k = b * _E + e
        i1 = sel_ref[k, 2].astype(jnp.int32)
        i2 = sel_ref[k, 3].astype(jnp.int32)
        la = f_ref[0, pl.ds(i1, 1)][0]
        lb = f_ref[0, pl.ds(i2, 1)][0]
        mixed = sel_ref[k, 0] * la + sel_ref[k, 1] * lb
        mixed = mixed.astype(jnp.bfloat16)

        h = jnp.dot(mixed, w1bf_ref[e],
                    preferred_element_type=jnp.float32) + b1_ref[e]
        h = jnp.maximum(h, 0.0).astype(jnp.bfloat16)
        y = jnp.dot(h, w2_ref[e],
                    preferred_element_type=jnp.float32)
        acc = y if acc is None else acc + y

    bsum = jnp.sum(b2_ref[...], axis=0)
    out_ref[0] = acc[1:, :] * (1.0 / _E) + bsum * (1.0 / _E)


def _resize_body(avg_ref, r_ref, rt_ref, out_ref):
    a1 = jnp.dot(avg_ref[0], rt_ref[...],
                 preferred_element_type=jnp.float32)
    r = r_ref[...]
    for c in range(_C):
        out_ref[0, c] = jnp.dot(r, a1[c * _H:(c + 1) * _H, :],
                                preferred_element_type=jnp.float32)


def kernel(features, gate_w, gate_b, w1, b1, w2, b2):
    B, L, T1, D = features.shape
    E = w1.shape[0]
    C = w2.shape[2]
    T = T1 - 1

    cls = features[:, :, 0, :]
    gwt = gate_w.T
    gb16 = jnp.pad(gate_b, (0, _LANES - E))
    sel = _sc_route()(cls, gwt, gb16)

    w2_bf = w2.astype(jnp.bfloat16)
    grid_spec = pltpu.PrefetchScalarGridSpec(
        num_scalar_prefetch=1,
        grid=(B,),
        in_specs=[
            pl.BlockSpec((1, L, T1, D), lambda b, s_s: (b, 0, 0, 0)),
            pl.BlockSpec((E, D, D), lambda b, s_s: (0, 0, 0)),
            pl.BlockSpec((E, 1, D), lambda b, s_s: (0, 0, 0)),
            pl.BlockSpec((E, D, C), lambda b, s_s: (0, 0, 0)),
            pl.BlockSpec((E, 1, C), lambda b, s_s: (0, 0, 0)),
        ],
        out_specs=pl.BlockSpec((1, T, C), lambda b, s_s: (b, 0, 0)),
        scratch_shapes=[pltpu.VMEM((E, D, D), jnp.bfloat16)],
    )
    avg = pl.pallas_call(
        _tc_body,
        grid_spec=grid_spec,
        out_shape=jax.ShapeDtypeStruct((B, T, C), jnp.float32),
    )(sel, features, w1, b1.reshape(E, 1, D),
      w2_bf, b2.reshape(E, 1, C))

    avg_m = avg.transpose(0, 2, 1).reshape(B, C * _H, _H)

    r_mat = jax.image.resize(jnp.eye(_H, dtype=jnp.float32), (_IMG, _H),
                             method="bilinear")

    out = pl.pallas_call(
        _resize_body,
        grid=(B,),
        in_specs=[
            pl.BlockSpec((1, C * _H, _H), lambda b: (b, 0, 0)),
            pl.BlockSpec((_IMG, _H), lambda b: (0, 0)),
            pl.BlockSpec((_H, _IMG), lambda b: (0, 0)),
        ],
        out_specs=pl.BlockSpec((1, C, _IMG, _IMG), lambda b: (b, 0, 0, 0)),
        out_shape=jax.ShapeDtypeStruct((B, C, _IMG, _IMG), jnp.float32),
    )(avg_m, r_mat, r_mat.T)
    return out
